# Initial kernel scaffold; baseline (speedup 1.0000x reference)
import jax
import jax.numpy as jnp
from jax.experimental import pallas as pl


def kernel(m_sim, d_sim, W_m, W_d, A1_W, A1_b, A2_W, train_mirna, train_disease):
    raise NotImplementedError("write your pallas kernel here")


# speedup vs baseline: 2.2463x; 2.2463x over previous
import jax
import jax.numpy as jnp
from jax.experimental import pallas as pl
from jax.experimental.pallas import tpu as pltpu

_NUM_M = 495
_NUM_D = 383
_DIM = 128
_LAYERS = 3
_N_EDGE = 5430

_MP = 512
_DP = 384
_EB = 512
_NB = -(-_N_EDGE // _EB)
_EP = _EB * _NB


def _body(m_sim_ref, d_sim_ref, WmT_ref, WdT_ref, A1mT_ref, A1dT_ref,
          A1b_ref, A2T_ref, tm_col_ref, td_col_ref, tm_row_ref,
          out_m_ref, out_d_ref):
    f32 = jnp.float32
    Em = jnp.dot(m_sim_ref[...], WmT_ref[...], preferred_element_type=f32)
    Ed = jnp.dot(d_sim_ref[...], WdT_ref[...], preferred_element_type=f32)

    A1mT = A1mT_ref[...]
    A1dT = A1dT_ref[...]
    A1b = A1b_ref[...]
    A2T = A2T_ref[...]

    S = jnp.zeros((_MP, _DP), f32)
    for b in range(_NB):
        tm_c = tm_col_ref[b]
        td_c = td_col_ref[b]
        tm_r = tm_row_ref[b]
        oh_m = (tm_c == jax.lax.broadcasted_iota(jnp.int32, (_EB, _MP), 1)
                ).astype(f32)
        oh_d = (td_c == jax.lax.broadcasted_iota(jnp.int32, (_EB, _DP), 1)
                ).astype(f32)
        me = jnp.dot(oh_m, Em, preferred_element_type=f32)
        de = jnp.dot(oh_d, Ed, preferred_element_type=f32)
        h = jnp.tanh(jnp.dot(jnp.maximum(me, 0.0), A1mT, preferred_element_type=f32)
                     + jnp.dot(jnp.maximum(de, 0.0), A1dT, preferred_element_type=f32)
                     + A1b)
        sc = jnp.dot(h, A2T, preferred_element_type=f32)
        eidx = b * _EB + jax.lax.broadcasted_iota(jnp.int32, (_EB, 1), 0)
        vals = jnp.exp(sc) * (eidx < _N_EDGE).astype(f32)
        oh_mT = (tm_r == jax.lax.broadcasted_iota(jnp.int32, (_MP, _EB), 0)
                 ).astype(f32)
        S = S + jnp.dot(oh_mT, oh_d * vals, preferred_element_type=f32)

    rowsum = jnp.sum(S, axis=1, keepdims=True)
    rm = jnp.where(rowsum > 0.0, 1.0 / rowsum, 0.0)
    ST = S.T
    colsum = jnp.sum(ST, axis=1, keepdims=True)
    rd = jnp.where(colsum > 0.0, 1.0 / colsum, 0.0)

    m_acc = jnp.zeros((_MP, _DIM), f32)
    d_acc = jnp.zeros((_DP, _DIM), f32)
    d_emb = Ed
    m_emb = Em
    for _ in range(_LAYERS):
        m_emb = jnp.tanh(jnp.dot(S, d_emb, preferred_element_type=f32) * rm)
        d_emb = jnp.tanh(jnp.dot(ST, m_emb, preferred_element_type=f32) * rd)
        m_acc = m_acc + m_emb
        d_acc = d_acc + d_emb
    out_m_ref[...] = m_acc
    out_d_ref[...] = d_acc


def kernel(m_sim, d_sim, W_m, W_d, A1_W, A1_b, A2_W, train_mirna, train_disease):
    f32 = jnp.float32
    m_sim_p = jnp.zeros((_MP, _MP), f32).at[:_NUM_M, :_NUM_M].set(m_sim)
    d_sim_p = jnp.zeros((_DP, _DP), f32).at[:_NUM_D, :_NUM_D].set(d_sim)
    WmT = jnp.zeros((_MP, _DIM), f32).at[:_NUM_M, :].set(W_m.T)
    WdT = jnp.zeros((_DP, _DIM), f32).at[:_NUM_D, :].set(W_d.T)
    A1mT = A1_W[:, :_DIM].T
    A1dT = A1_W[:, _DIM:].T
    A1b = A1_b.reshape(1, _DIM)
    A2T = A2_W.T

    tm = (train_mirna - _NUM_D).astype(jnp.int32)
    td = train_disease.astype(jnp.int32)
    tm_p = jnp.zeros((_EP,), jnp.int32).at[:_N_EDGE].set(tm)
    td_p = jnp.zeros((_EP,), jnp.int32).at[:_N_EDGE].set(td)
    tm_col = tm_p.reshape(_NB, _EB, 1)
    td_col = td_p.reshape(_NB, _EB, 1)
    tm_row = tm_p.reshape(_NB, 1, _EB)

    out_m, out_d = pl.pallas_call(
        _body,
        out_shape=(
            jax.ShapeDtypeStruct((_MP, _DIM), f32),
            jax.ShapeDtypeStruct((_DP, _DIM), f32),
        ),
    )(m_sim_p, d_sim_p, WmT, WdT, A1mT, A1dT, A1b, A2T,
      tm_col, td_col, tm_row)
    return (out_m[:_NUM_M], out_d[:_NUM_D])


# speedup vs baseline: 3.1784x; 1.4149x over previous
import jax
import jax.numpy as jnp
from jax.experimental import pallas as pl
from jax.experimental.pallas import tpu as pltpu

_NUM_M = 495
_NUM_D = 383
_DIM = 128
_LAYERS = 3
_N_EDGE = 5430

_EB = 512
_NB = -(-_N_EDGE // _EB)
_EP = _EB * _NB

_RT2 = (((1,), (1,)), ((), ()))


def _body(m_sim_ref, d_sim_ref, Wm_ref, Wd_ref, A1W_ref,
          A1b_ref, A2W_ref, tm_col_ref, td_col_ref, tm_row_ref,
          out_m_ref, out_d_ref):
    f32 = jnp.float32
    dg = jax.lax.dot_general
    Em = dg(m_sim_ref[...], Wm_ref[...], _RT2, preferred_element_type=f32)
    Ed = dg(d_sim_ref[...], Wd_ref[...], _RT2, preferred_element_type=f32)

    A1m = A1W_ref[:, :_DIM]
    A1d = A1W_ref[:, _DIM:]
    A1b = A1b_ref[...]
    A2 = A2W_ref[...]

    S = jnp.zeros((_NUM_M, _NUM_D), f32)
    for b in range(_NB):
        tm_c = tm_col_ref[b]
        td_c = td_col_ref[b]
        tm_r = tm_row_ref[b]
        oh_m = (tm_c == jax.lax.broadcasted_iota(jnp.int32, (_EB, _NUM_M), 1)
                ).astype(f32)
        oh_d = (td_c == jax.lax.broadcasted_iota(jnp.int32, (_EB, _NUM_D), 1)
                ).astype(f32)
        me = jnp.dot(oh_m, Em, preferred_element_type=f32)
        de = jnp.dot(oh_d, Ed, preferred_element_type=f32)
        h = jnp.tanh(dg(jnp.maximum(me, 0.0), A1m, _RT2, preferred_element_type=f32)
                     + dg(jnp.maximum(de, 0.0), A1d, _RT2, preferred_element_type=f32)
                     + A1b)
        sc = dg(h, A2, _RT2, preferred_element_type=f32)
        eidx = b * _EB + jax.lax.broadcasted_iota(jnp.int32, (_EB, 1), 0)
        vals = jnp.exp(sc) * (eidx < _N_EDGE).astype(f32)
        oh_mT = (tm_r == jax.lax.broadcasted_iota(jnp.int32, (_NUM_M, _EB), 0)
                 ).astype(f32)
        S = S + jnp.dot(oh_mT, oh_d * vals, preferred_element_type=f32)

    rowsum = jnp.sum(S, axis=1, keepdims=True)
    rm = jnp.where(rowsum > 0.0, 1.0 / rowsum, 0.0)
    ST = S.T
    colsum = jnp.sum(ST, axis=1, keepdims=True)
    rd = jnp.where(colsum > 0.0, 1.0 / colsum, 0.0)

    m_acc = jnp.zeros((_NUM_M, _DIM), f32)
    d_acc = jnp.zeros((_NUM_D, _DIM), f32)
    d_emb = Ed
    for _ in range(_LAYERS):
        m_emb = jnp.tanh(jnp.dot(S, d_emb, preferred_element_type=f32) * rm)
        d_emb = jnp.tanh(jnp.dot(ST, m_emb, preferred_element_type=f32) * rd)
        m_acc = m_acc + m_emb
        d_acc = d_acc + d_emb
    out_m_ref[...] = m_acc
    out_d_ref[...] = d_acc


def kernel(m_sim, d_sim, W_m, W_d, A1_W, A1_b, A2_W, train_mirna, train_disease):
    f32 = jnp.float32
    tm = (train_mirna - _NUM_D).astype(jnp.int32)
    td = train_disease.astype(jnp.int32)
    pad = (0, _EP - _N_EDGE)
    tm_p = jnp.pad(tm, pad)
    td_p = jnp.pad(td, pad)
    tm_col = tm_p.reshape(_NB, _EB, 1)
    td_col = td_p.reshape(_NB, _EB, 1)
    tm_row = tm_p.reshape(_NB, 1, _EB)

    out_m, out_d = pl.pallas_call(
        _body,
        out_shape=(
            jax.ShapeDtypeStruct((_NUM_M, _DIM), f32),
            jax.ShapeDtypeStruct((_NUM_D, _DIM), f32),
        ),
    )(m_sim, d_sim, W_m, W_d, A1_W, A1_b.reshape(1, _DIM), A2_W,
      tm_col, td_col, tm_row)
    return (out_m, out_d)


# speedup vs baseline: 5.3225x; 1.6746x over previous
import jax
import jax.numpy as jnp
from jax.experimental import pallas as pl
from jax.experimental.pallas import tpu as pltpu

_NUM_M = 495
_NUM_D = 383
_OFF = 383
_DIM = 128
_LAYERS = 3
_N_EDGE = 5430
_EB = 512

_RT2 = (((1,), (1,)), ((), ()))


def _body(m_sim_ref, d_sim_ref, Wm_ref, Wd_ref, A1W_ref,
          A1b_ref, A2W_ref, tm_ref, td_ref,
          out_m_ref, out_d_ref):
    f32 = jnp.float32
    dg = jax.lax.dot_general
    Em = dg(m_sim_ref[...], Wm_ref[...], _RT2, preferred_element_type=f32)
    Ed = dg(d_sim_ref[...], Wd_ref[...], _RT2, preferred_element_type=f32)
    EmT = Em.T
    EdT = Ed.T

    A1m = A1W_ref[:, :_DIM]
    A1d = A1W_ref[:, _DIM:]
    A1b_col = A1b_ref[...].reshape(1, _DIM).T
    A2 = A2W_ref[...]

    S = jnp.zeros((_NUM_M, _NUM_D), f32)
    for start in range(0, _N_EDGE, _EB):
        nb = min(_EB, _N_EDGE - start)
        tm_r = tm_ref[start:start + nb].reshape(1, nb)
        td_r = td_ref[start:start + nb].reshape(1, nb)
        oh_mT = (tm_r == _OFF + jax.lax.broadcasted_iota(jnp.int32, (_NUM_M, nb), 0)
                 ).astype(f32)
        oh_dT = (td_r == jax.lax.broadcasted_iota(jnp.int32, (_NUM_D, nb), 0)
                 ).astype(f32)
        meT = jnp.dot(EmT, oh_mT, preferred_element_type=f32)
        deT = jnp.dot(EdT, oh_dT, preferred_element_type=f32)
        hT = jnp.tanh(jnp.dot(A1m, jnp.maximum(meT, 0.0), preferred_element_type=f32)
                      + jnp.dot(A1d, jnp.maximum(deT, 0.0), preferred_element_type=f32)
                      + A1b_col)
        sc = jnp.dot(A2, hT, preferred_element_type=f32)
        vals = jnp.exp(sc)
        S = S + dg(oh_mT, oh_dT * vals, _RT2, preferred_element_type=f32)

    rowsum = jnp.sum(S, axis=1, keepdims=True)
    rm = jnp.where(rowsum > 0.0, 1.0 / rowsum, 0.0)
    ST = S.T
    colsum = jnp.sum(ST, axis=1, keepdims=True)
    rd = jnp.where(colsum > 0.0, 1.0 / colsum, 0.0)

    m_acc = jnp.zeros((_NUM_M, _DIM), f32)
    d_acc = jnp.zeros((_NUM_D, _DIM), f32)
    d_emb = Ed
    for _ in range(_LAYERS):
        m_emb = jnp.tanh(jnp.dot(S, d_emb, preferred_element_type=f32) * rm)
        d_emb = jnp.tanh(jnp.dot(ST, m_emb, preferred_element_type=f32) * rd)
        m_acc = m_acc + m_emb
        d_acc = d_acc + d_emb
    out_m_ref[...] = m_acc
    out_d_ref[...] = d_acc


def kernel(m_sim, d_sim, W_m, W_d, A1_W, A1_b, A2_W, train_mirna, train_disease):
    f32 = jnp.float32
    return pl.pallas_call(
        _body,
        out_shape=(
            jax.ShapeDtypeStruct((_NUM_M, _DIM), f32),
            jax.ShapeDtypeStruct((_NUM_D, _DIM), f32),
        ),
    )(m_sim, d_sim, W_m, W_d, A1_W, A1_b, A2_W, train_mirna, train_disease)
